# baseline (speedup 1.0000x reference)
import jax
import jax.numpy as jnp
from jax.experimental import pallas as pl
from jax.experimental.pallas import tpu as pltpu


VOCAB = 8388608
LANES = 128
ROWS = VOCAB // LANES
BLOCK_ROWS = 16384
STEPS = ROWS // BLOCK_ROWS

NT = 65536
T_ROWS = NT // LANES


def _dense_neg_kernel(p_ref, acc_ref):
    s = pl.program_id(0)

    @pl.when(s == 0)
    def _init():
        acc_ref[...] = jnp.zeros_like(acc_ref)

    p = p_ref[...]
    l = jnp.log(1.0 - p + 1e-15)
    acc_ref[...] += jnp.sum(l, axis=0)[None, :]


def _finalize_kernel(s_ref, prev_ref, pv_ref, nall_ref, p0_ref, out_ref):
    sid = s_ref[...]
    mask = jnp.logical_and(sid != prev_ref[...], sid > 0)
    pv = pv_ref[...]
    zero = jnp.zeros_like(pv)
    lp = jnp.where(mask, jnp.log(pv + 1e-30), zero)
    ln = jnp.where(mask, jnp.log(1.0 - pv + 1e-15), zero)
    lp_s = jnp.sum(lp, keepdims=True)
    ln_s = jnp.sum(ln, keepdims=True)
    np_s = jnp.sum(jnp.where(mask, 1.0, 0.0), keepdims=True)

    p0 = p0_ref[0, 0]
    nall_s = jnp.sum(nall_ref[...], keepdims=True) - jnp.log(1.0 - p0 + 1e-15)

    neg_s = nall_s - ln_s
    n_neg = jnp.float32(VOCAB - 1) - np_s
    out_ref[...] = -lp_s / np_s - neg_s / n_neg


def kernel(input_probs, target):
    probs_flat = input_probs.reshape(-1)
    probs2d = input_probs.reshape(ROWS, LANES)

    tgt = target.reshape(-1).astype(jnp.int32)
    s = jnp.sort(tgt.astype(jnp.float32)).astype(jnp.int32)
    pv = probs_flat[s]
    prev = jnp.concatenate([jnp.full((1,), -1, jnp.int32), s[:-1]])

    s2d = s.reshape(T_ROWS, LANES)
    prev2d = prev.reshape(T_ROWS, LANES)
    pv2d = pv.reshape(T_ROWS, LANES)

    nall = pl.pallas_call(
        _dense_neg_kernel,
        grid=(STEPS,),
        in_specs=[pl.BlockSpec((BLOCK_ROWS, LANES), lambda i: (i, 0))],
        out_specs=pl.BlockSpec((1, LANES), lambda i: (0, 0)),
        out_shape=jax.ShapeDtypeStruct((1, LANES), jnp.float32),
        compiler_params=pltpu.CompilerParams(
            dimension_semantics=("arbitrary",)),
    )(probs2d)

    out = pl.pallas_call(
        _finalize_kernel,
        grid=(1,),
        in_specs=[
            pl.BlockSpec((T_ROWS, LANES), lambda i: (0, 0)),
            pl.BlockSpec((T_ROWS, LANES), lambda i: (0, 0)),
            pl.BlockSpec((T_ROWS, LANES), lambda i: (0, 0)),
            pl.BlockSpec((1, LANES), lambda i: (0, 0)),
            pl.BlockSpec((8, LANES), lambda i: (0, 0)),
        ],
        out_specs=pl.BlockSpec((1, 1), lambda i: (0, 0)),
        out_shape=jax.ShapeDtypeStruct((1, 1), jnp.float32),
        compiler_params=pltpu.CompilerParams(
            dimension_semantics=("arbitrary",)),
    )(s2d, prev2d, pv2d, nall, probs2d)

    return out[0, 0]


# speedup vs baseline: 1.1721x; 1.1721x over previous
import jax
import jax.numpy as jnp
from jax.experimental import pallas as pl
from jax.experimental.pallas import tpu as pltpu


VOCAB = 8388608
LANES = 128
ROWS = VOCAB // LANES
BLOCK_ROWS = 16384
STEPS = ROWS // BLOCK_ROWS

NT = 65536
T_ROWS = NT // LANES


def _dense_neg_kernel(p_ref, acc_ref):
    s = pl.program_id(0)

    @pl.when(s == 0)
    def _init():
        acc_ref[...] = jnp.zeros_like(acc_ref)

    p = p_ref[...]
    l = jnp.log(1.0 - p + 1e-15)
    acc_ref[...] += jnp.sum(l, axis=0)[None, :]


def _finalize_kernel(s_ref, prev_ref, pv_ref, nall_ref, p0_ref, out_ref):
    sid = s_ref[...]
    mask = jnp.logical_and(sid != prev_ref[...], sid > 0)
    pv = pv_ref[...]
    zero = jnp.zeros_like(pv)
    lp = jnp.where(mask, jnp.log(pv + 1e-30), zero)
    ln = jnp.where(mask, jnp.log(1.0 - pv + 1e-15), zero)
    lp_s = jnp.sum(lp, keepdims=True)
    ln_s = jnp.sum(ln, keepdims=True)
    np_s = jnp.sum(jnp.where(mask, 1.0, 0.0), keepdims=True)

    p0 = p0_ref[0, 0]
    nall_s = jnp.sum(nall_ref[...], keepdims=True) - jnp.log(1.0 - p0 + 1e-15)

    neg_s = nall_s - ln_s
    n_neg = jnp.float32(VOCAB - 1) - np_s
    out_ref[...] = -lp_s / np_s - neg_s / n_neg


def kernel(input_probs, target):
    probs_flat = input_probs.reshape(-1)
    probs2d = input_probs.reshape(ROWS, LANES)

    tgt = target.reshape(-1).astype(jnp.int32)
    s = jnp.sort(tgt)
    pv = probs_flat[s]
    prev = jnp.concatenate([jnp.full((1,), -1, jnp.int32), s[:-1]])

    s2d = s.reshape(T_ROWS, LANES)
    prev2d = prev.reshape(T_ROWS, LANES)
    pv2d = pv.reshape(T_ROWS, LANES)

    nall = pl.pallas_call(
        _dense_neg_kernel,
        grid=(STEPS,),
        in_specs=[pl.BlockSpec((BLOCK_ROWS, LANES), lambda i: (i, 0))],
        out_specs=pl.BlockSpec((1, LANES), lambda i: (0, 0)),
        out_shape=jax.ShapeDtypeStruct((1, LANES), jnp.float32),
        compiler_params=pltpu.CompilerParams(
            dimension_semantics=("arbitrary",)),
    )(probs2d)

    out = pl.pallas_call(
        _finalize_kernel,
        grid=(1,),
        in_specs=[
            pl.BlockSpec((T_ROWS, LANES), lambda i: (0, 0)),
            pl.BlockSpec((T_ROWS, LANES), lambda i: (0, 0)),
            pl.BlockSpec((T_ROWS, LANES), lambda i: (0, 0)),
            pl.BlockSpec((1, LANES), lambda i: (0, 0)),
            pl.BlockSpec((8, LANES), lambda i: (0, 0)),
        ],
        out_specs=pl.BlockSpec((1, 1), lambda i: (0, 0)),
        out_shape=jax.ShapeDtypeStruct((1, 1), jnp.float32),
        compiler_params=pltpu.CompilerParams(
            dimension_semantics=("arbitrary",)),
    )(s2d, prev2d, pv2d, nall, probs2d)

    return out[0, 0]


# speedup vs baseline: 1.3190x; 1.1253x over previous
import jax
import jax.numpy as jnp
from jax.experimental import pallas as pl
from jax.experimental.pallas import tpu as pltpu


VOCAB = 8388608
LANES = 128
ROWS = VOCAB // LANES
BLOCK_ROWS = 16384
STEPS = ROWS // BLOCK_ROWS

NT = 65536
T_ROWS = NT // LANES


def _dense_neg_kernel(p_ref, acc_ref):
    s = pl.program_id(0)

    @pl.when(s == 0)
    def _init():
        acc_ref[...] = jnp.zeros_like(acc_ref)

    p = p_ref[...]
    l = jnp.log(1.0 - p + 1e-15)
    acc_ref[...] += jnp.sum(l, axis=0)[None, :]


def _finalize_kernel(s_ref, prev_ref, pv_ref, nall_ref, p0_ref, out_ref):
    sid = s_ref[...]
    mask = jnp.logical_and(sid != prev_ref[...], sid > 0)
    pv = pv_ref[...]
    zero = jnp.zeros_like(pv)
    lp = jnp.where(mask, jnp.log(pv + 1e-30), zero)
    ln = jnp.where(mask, jnp.log(1.0 - pv + 1e-15), zero)
    lp_s = jnp.sum(lp, keepdims=True)
    ln_s = jnp.sum(ln, keepdims=True)
    np_s = jnp.sum(jnp.where(mask, 1.0, 0.0), keepdims=True)

    p0 = p0_ref[0, 0]
    nall_s = jnp.sum(nall_ref[...], keepdims=True) - jnp.log(1.0 - p0 + 1e-15)

    neg_s = nall_s - ln_s
    n_neg = jnp.float32(VOCAB - 1) - np_s
    out_ref[...] = -lp_s / np_s - neg_s / n_neg


def kernel(input_probs, target):
    probs_flat = input_probs.reshape(-1)
    probs2d = input_probs.reshape(ROWS, LANES)

    tgt = target.reshape(-1).astype(jnp.int32)
    pv_raw = probs_flat[tgt]
    s, pv = jax.lax.sort((tgt, pv_raw), num_keys=1, is_stable=False)
    prev = jnp.concatenate([jnp.full((1,), -1, jnp.int32), s[:-1]])

    s2d = s.reshape(T_ROWS, LANES)
    prev2d = prev.reshape(T_ROWS, LANES)
    pv2d = pv.reshape(T_ROWS, LANES)

    nall = pl.pallas_call(
        _dense_neg_kernel,
        grid=(STEPS,),
        in_specs=[pl.BlockSpec((BLOCK_ROWS, LANES), lambda i: (i, 0))],
        out_specs=pl.BlockSpec((1, LANES), lambda i: (0, 0)),
        out_shape=jax.ShapeDtypeStruct((1, LANES), jnp.float32),
        compiler_params=pltpu.CompilerParams(
            dimension_semantics=("arbitrary",)),
    )(probs2d)

    out = pl.pallas_call(
        _finalize_kernel,
        grid=(1,),
        in_specs=[
            pl.BlockSpec((T_ROWS, LANES), lambda i: (0, 0)),
            pl.BlockSpec((T_ROWS, LANES), lambda i: (0, 0)),
            pl.BlockSpec((T_ROWS, LANES), lambda i: (0, 0)),
            pl.BlockSpec((1, LANES), lambda i: (0, 0)),
            pl.BlockSpec((8, LANES), lambda i: (0, 0)),
        ],
        out_specs=pl.BlockSpec((1, 1), lambda i: (0, 0)),
        out_shape=jax.ShapeDtypeStruct((1, 1), jnp.float32),
        compiler_params=pltpu.CompilerParams(
            dimension_semantics=("arbitrary",)),
    )(s2d, prev2d, pv2d, nall, probs2d)

    return out[0, 0]


# speedup vs baseline: 1.3338x; 1.0113x over previous
import jax
import jax.numpy as jnp
from jax.experimental import pallas as pl
from jax.experimental.pallas import tpu as pltpu


VOCAB = 8388608
LANES = 128
ROWS = VOCAB // LANES
BLOCK_ROWS = 16384
STEPS = ROWS // BLOCK_ROWS

NT = 65536
T_ROWS = NT // LANES


def _dense_neg_kernel(p_ref, acc_ref):
    s = pl.program_id(0)

    @pl.when(s == 0)
    def _init():
        acc_ref[...] = jnp.zeros_like(acc_ref)

    p = p_ref[...]
    l = jnp.log(1.0 - p + 1e-15)
    acc_ref[...] += jnp.sum(l, axis=0)[None, :]


def _finalize_kernel(s_ref, pv_ref, nall_ref, p0_ref, out_ref):
    sid = s_ref[...]
    rolled = pltpu.roll(sid, 1, 1)
    c127 = pltpu.roll(sid[:, 127:128], 1, 0)
    lanes = jax.lax.broadcasted_iota(jnp.int32, sid.shape, 1)
    rows = jax.lax.broadcasted_iota(jnp.int32, sid.shape, 0)
    prev = jnp.where(lanes == 0, c127, rolled)
    first = jnp.logical_and(rows == 0, lanes == 0)
    mask = jnp.logical_and(jnp.logical_or(sid != prev, first), sid > 0)
    pv = pv_ref[...]
    zero = jnp.zeros_like(pv)
    lp = jnp.where(mask, jnp.log(pv + 1e-30), zero)
    ln = jnp.where(mask, jnp.log(1.0 - pv + 1e-15), zero)
    lp_s = jnp.sum(lp, keepdims=True)
    ln_s = jnp.sum(ln, keepdims=True)
    np_s = jnp.sum(jnp.where(mask, 1.0, 0.0), keepdims=True)

    p0 = p0_ref[0, 0]
    nall_s = jnp.sum(nall_ref[...], keepdims=True) - jnp.log(1.0 - p0 + 1e-15)

    neg_s = nall_s - ln_s
    n_neg = jnp.float32(VOCAB - 1) - np_s
    out_ref[...] = -lp_s / np_s - neg_s / n_neg


def kernel(input_probs, target):
    probs_flat = input_probs.reshape(-1)
    probs2d = input_probs.reshape(ROWS, LANES)

    tgt = target.reshape(-1).astype(jnp.int32)
    pv_raw = probs_flat[tgt]
    s, pv = jax.lax.sort((tgt, pv_raw), num_keys=1, is_stable=False)

    s2d = s.reshape(T_ROWS, LANES)
    pv2d = pv.reshape(T_ROWS, LANES)

    nall = pl.pallas_call(
        _dense_neg_kernel,
        grid=(STEPS,),
        in_specs=[pl.BlockSpec((BLOCK_ROWS, LANES), lambda i: (i, 0))],
        out_specs=pl.BlockSpec((1, LANES), lambda i: (0, 0)),
        out_shape=jax.ShapeDtypeStruct((1, LANES), jnp.float32),
        compiler_params=pltpu.CompilerParams(
            dimension_semantics=("arbitrary",)),
    )(probs2d)

    out = pl.pallas_call(
        _finalize_kernel,
        grid=(1,),
        in_specs=[
            pl.BlockSpec((T_ROWS, LANES), lambda i: (0, 0)),
            pl.BlockSpec((T_ROWS, LANES), lambda i: (0, 0)),
            pl.BlockSpec((1, LANES), lambda i: (0, 0)),
            pl.BlockSpec((8, LANES), lambda i: (0, 0)),
        ],
        out_specs=pl.BlockSpec((1, 1), lambda i: (0, 0)),
        out_shape=jax.ShapeDtypeStruct((1, 1), jnp.float32),
        compiler_params=pltpu.CompilerParams(
            dimension_semantics=("arbitrary",)),
    )(s2d, pv2d, nall, probs2d)

    return out[0, 0]
